# baseline (speedup 1.0000x reference)
import functools

import jax
import jax.numpy as jnp
from jax import lax
from jax.experimental import pallas as pl
from jax.experimental.pallas import tpu as pltpu
from jax.experimental.pallas import tpu_sc as plsc

NC = 2
NS = 16
NW = NC * NS
L = 16

CB = 128

SHARE0_NUM, SHARE0_DEN = 1, 2


def _acc_rows(n_nodes):
    zrows = -(-(-(-(n_nodes + 1) // NS)) // CB) * CB
    return NS * zrows, zrows


def _sc_aggregate(n_nodes, d, pair_chunks, c0, with_deg):
    npad, zrows = _acc_rows(n_nodes)

    grp = 8
    assert pair_chunks % grp == 0 and c0 % grp == 0 and (pair_chunks - c0) % grp == 0
    out_type = [jax.ShapeDtypeStruct((NC, npad, d), jnp.float32)]
    scratch = [
        pltpu.VMEM((grp, CB), jnp.int32),
        pltpu.VMEM((grp, CB), jnp.int32),
        pltpu.VMEM((2, CB, d // 2), jnp.int32),
        pltpu.VMEM((CB, d), jnp.float32),
        pltpu.VMEM_SHARED((npad, d), jnp.float32),
        pltpu.SemaphoreType.DMA,
        pltpu.SemaphoreType.DMA,
    ]
    if with_deg:
        out_type.append(jax.ShapeDtypeStruct((NC, npad, L), jnp.float32))
        scratch += [
            pltpu.VMEM((CB, L), jnp.float32),
            pltpu.VMEM_SHARED((npad, L), jnp.float32),
        ]

    mesh = plsc.VectorSubcoreMesh(core_axis_name="c", subcore_axis_name="s")

    def body(x_hbm, src_hbm, dst_hbm, *rest):
        if with_deg:
            (agg_out, deg_out, srcv, dstv, rows_g, rows, aggs, sem_a, sem_b,
             ones_v, degs) = rest
        else:
            agg_out, srcv, dstv, rows_g, rows, aggs, sem_a, sem_b = rest
        cid = lax.axis_index("c")
        sid = lax.axis_index("s")
        base = sid * pair_chunks + cid * c0
        ngroups = lax.select(cid == 0, c0 // grp, (pair_chunks - c0) // grp)

        def zrow(r, _):
            for c in range(d // L):
                rows[r, pl.ds(c * L, L)] = jnp.zeros((L,), jnp.float32)
            return _
        lax.fori_loop(0, CB, zrow, 0)
        for k in range(zrows // CB):
            pltpu.sync_copy(rows, aggs.at[pl.ds(sid * zrows + k * CB, CB)])

        if with_deg:
            def zdeg(r, _):
                ones_v[r, pl.ds(0, L)] = jnp.zeros((L,), jnp.float32)
                return _
            lax.fori_loop(0, CB, zdeg, 0)
            for k in range(zrows // CB):
                pltpu.sync_copy(ones_v, degs.at[pl.ds(sid * zrows + k * CB, CB)])

            def fill(r, _):
                ones_v[r, pl.ds(0, L)] = jnp.full((L,), 1.0, jnp.float32)
                return _
            lax.fori_loop(0, CB, fill, 0)

        plsc.subcore_barrier()


        def convert(b):
            def conv(r, _):
                for u in range(2):
                    for k in range(d // 32):
                        v = rows_g[b, 2 * r + u, pl.ds(k * L, L)]
                        lo = plsc.bitcast(lax.shift_left(v, 16), jnp.float32)
                        hi = plsc.bitcast(
                            lax.bitwise_and(v, jnp.int32(-65536)), jnp.float32)
                        rows[2 * r + u, pl.ds(k * L, L)] = lo
                        rows[2 * r + u, pl.ds(d // 2 + k * L, L)] = hi
                return _
            lax.fori_loop(0, CB // 2, conv, 0)

        gat_sems = (sem_a, sem_b)

        def group(g, _):
            cb = base + g * grp
            pltpu.sync_copy(src_hbm.at[pl.ds(cb, grp)], srcv)
            pltpu.sync_copy(dst_hbm.at[pl.ds(cb, grp)], dstv)
            gat = {0: pltpu.async_copy(x_hbm.at[srcv.at[0]], rows_g.at[0], sem_a)}
            for j in range(grp):
                b = j & 1
                if j + 1 < grp:
                    gat[j + 1] = pltpu.async_copy(
                        x_hbm.at[srcv.at[j + 1]], rows_g.at[1 - b],
                        gat_sems[1 - b])
                gat[j].wait()
                convert(b)
                pltpu.sync_copy(rows, aggs.at[dstv.at[j]], add=True)
                if with_deg:
                    pltpu.sync_copy(ones_v, degs.at[dstv.at[j]], add=True)
            return _
        lax.fori_loop(0, ngroups, group, 0)

        plsc.subcore_barrier()

        for k in range(zrows // CB):
            off = sid * zrows + k * CB
            pltpu.sync_copy(aggs.at[pl.ds(off, CB)], rows)
            pltpu.sync_copy(rows, agg_out.at[cid, pl.ds(off, CB)])
        if with_deg:
            for k in range(zrows // CB):
                off = sid * zrows + k * CB
                pltpu.sync_copy(degs.at[pl.ds(off, CB)], ones_v)
                pltpu.sync_copy(ones_v, deg_out.at[cid, pl.ds(off, CB)])

    return pl.kernel(
        body, out_type=out_type, mesh=mesh, scratch_types=scratch,
        compiler_params=pltpu.CompilerParams(
            use_tc_tiling_on_sc=False, needs_layout_passes=False))


def _pack_bf16(x):
    d = x.shape[1]
    lo = lax.bitcast_convert_type(x[:, :d // 2].astype(jnp.bfloat16), jnp.uint16)
    hi = lax.bitcast_convert_type(x[:, d // 2:].astype(jnp.bfloat16), jnp.uint16)
    return (hi.astype(jnp.int32) << 16) | lo.astype(jnp.int32)


def _tc_layer(agg, deg, x, wt, wb, b):
    n, d = x.shape

    def body(agg_ref, deg_ref, x_ref, wt_ref, wb_ref, b_ref, o_ref, op_ref):
        s = agg_ref[0, :n] + agg_ref[1, :n]
        cnt = deg_ref[0, :n, 0:1] + deg_ref[1, :n, 0:1]
        s = s / jnp.maximum(cnt, 1.0)
        h = (jnp.dot(s, wt_ref[...], preferred_element_type=jnp.float32)
             + jnp.dot(x_ref[...], wb_ref[...], preferred_element_type=jnp.float32)
             + b_ref[...])
        h = jnp.maximum(h, 0.0)
        nrm = jnp.sqrt(jnp.sum(h * h, axis=1, keepdims=True))
        h = h / jnp.maximum(nrm, 1e-12)
        h = jnp.maximum(h, 0.0)
        o_ref[...] = h
        lo = lax.bitcast_convert_type(
            h[:, :d // 2].astype(jnp.bfloat16), jnp.uint16)
        hi = lax.bitcast_convert_type(
            h[:, d // 2:].astype(jnp.bfloat16), jnp.uint16)
        op_ref[...] = (hi.astype(jnp.int32) << 16) | lo.astype(jnp.int32)

    return pl.pallas_call(
        body,
        out_shape=[jax.ShapeDtypeStruct((n, d), jnp.float32),
                   jax.ShapeDtypeStruct((n, d // 2), jnp.int32)],
    )(agg, deg, x, wt, wb, b.reshape(1, -1))


def _tc_final(agg, deg, x, wt, wb, b, wp1, bp1, wp2, bp2):
    n, d = x.shape
    d_out = wp2.shape[1]

    def body(agg_ref, deg_ref, x_ref, wt_ref, wb_ref, b_ref,
             wp1_ref, bp1_ref, wp2_ref, bp2_ref, o_ref):
        s = agg_ref[0, :n] + agg_ref[1, :n]
        cnt = deg_ref[0, :n, 0:1] + deg_ref[1, :n, 0:1]
        s = s / jnp.maximum(cnt, 1.0)
        h = (jnp.dot(s, wt_ref[...], preferred_element_type=jnp.float32)
             + jnp.dot(x_ref[...], wb_ref[...], preferred_element_type=jnp.float32)
             + b_ref[...])
        h = jnp.maximum(h, 0.0)
        nrm = jnp.sqrt(jnp.sum(h * h, axis=1, keepdims=True))
        h = h / jnp.maximum(nrm, 1e-12)
        h = jnp.maximum(h, 0.0)
        p = jnp.dot(h, wp1_ref[...], preferred_element_type=jnp.float32) + bp1_ref[...]
        q = jnp.dot(p, wp2_ref[...], preferred_element_type=jnp.float32) + bp2_ref[...]
        m = jnp.max(q, axis=1, keepdims=True)
        e = jnp.exp(q - m)
        lse = jnp.log(jnp.sum(e, axis=1, keepdims=True))
        o_ref[...] = q - m - lse

    return pl.pallas_call(
        body,
        out_shape=jax.ShapeDtypeStruct((n, d_out), jnp.float32),
    )(agg, deg, x, wt, wb, b.reshape(1, -1),
      wp1, bp1.reshape(1, -1), wp2, bp2.reshape(1, -1))


def kernel(x, edge_index, batch, W0, b0, W1, b1, Wp1, bp1, Wp2, bp2):
    n, d = x.shape
    e = edge_index.shape[1]
    pair_chunks = -(-(-(-e // (NS * CB))) // 8) * 8
    epad = NS * pair_chunks * CB
    c0 = (pair_chunks * SHARE0_NUM // SHARE0_DEN) // 8 * 8

    src = edge_index[0]
    dst = edge_index[1]
    npad, _ = _acc_rows(n)
    pad_dst = n + jnp.arange(epad - e, dtype=jnp.int32) % (npad - n)
    srcp = jnp.concatenate(
        [src, jnp.zeros((epad - e,), jnp.int32)]).reshape(-1, CB)
    dstp = jnp.concatenate([dst, pad_dst]).reshape(-1, CB)

    agg_deg = _sc_aggregate(n, d, pair_chunks, c0, True)
    agg_only = _sc_aggregate(n, d, pair_chunks, c0, False)

    agg0, deg = agg_deg(_pack_bf16(x), srcp, dstp)
    h1, h1p = _tc_layer(agg0, deg, x, W0[:d], W0[d:], b0)
    (agg1,) = agg_only(h1p, srcp, dstp)
    return _tc_final(agg1, deg, h1, W1[:d], W1[d:], b1, Wp1, bp1, Wp2, bp2)


# speedup vs baseline: 1.0682x; 1.0682x over previous
import functools

import jax
import jax.numpy as jnp
from jax import lax
from jax.experimental import pallas as pl
from jax.experimental.pallas import tpu as pltpu
from jax.experimental.pallas import tpu_sc as plsc

NC = 2
NS = 16
NW = NC * NS
L = 16

CB = 128

SHARE0_NUM, SHARE0_DEN = 11, 20


def _acc_rows(n_nodes):
    zrows = -(-(-(-(n_nodes + 1) // NS)) // CB) * CB
    return NS * zrows, zrows


def _sc_aggregate(n_nodes, d, pair_chunks, c0, with_deg):
    npad, zrows = _acc_rows(n_nodes)

    grp = 8
    assert pair_chunks % grp == 0 and c0 % grp == 0 and (pair_chunks - c0) % grp == 0
    out_type = [jax.ShapeDtypeStruct((NC, npad, d), jnp.float32)]
    scratch = [
        pltpu.VMEM((grp, CB), jnp.int32),
        pltpu.VMEM((grp, CB), jnp.int32),
        pltpu.VMEM((2, CB, d // 2), jnp.int32),
        pltpu.VMEM((CB, d), jnp.float32),
        pltpu.VMEM_SHARED((npad, d), jnp.float32),
        pltpu.SemaphoreType.DMA,
        pltpu.SemaphoreType.DMA,
    ]
    if with_deg:
        out_type.append(jax.ShapeDtypeStruct((NC, npad, L), jnp.float32))
        scratch += [
            pltpu.VMEM((CB, L), jnp.float32),
            pltpu.VMEM_SHARED((npad, L), jnp.float32),
        ]

    mesh = plsc.VectorSubcoreMesh(core_axis_name="c", subcore_axis_name="s")

    def body(x_hbm, src_hbm, dst_hbm, *rest):
        if with_deg:
            (agg_out, deg_out, srcv, dstv, rows_g, rows, aggs, sem_a, sem_b,
             ones_v, degs) = rest
        else:
            agg_out, srcv, dstv, rows_g, rows, aggs, sem_a, sem_b = rest
        cid = lax.axis_index("c")
        sid = lax.axis_index("s")
        base = sid * pair_chunks + cid * c0
        ngroups = lax.select(cid == 0, c0 // grp, (pair_chunks - c0) // grp)

        def zrow(r, _):
            for c in range(d // L):
                rows[r, pl.ds(c * L, L)] = jnp.zeros((L,), jnp.float32)
            return _
        lax.fori_loop(0, CB, zrow, 0)
        for k in range(zrows // CB):
            pltpu.sync_copy(rows, aggs.at[pl.ds(sid * zrows + k * CB, CB)])

        if with_deg:
            def zdeg(r, _):
                ones_v[r, pl.ds(0, L)] = jnp.zeros((L,), jnp.float32)
                return _
            lax.fori_loop(0, CB, zdeg, 0)
            for k in range(zrows // CB):
                pltpu.sync_copy(ones_v, degs.at[pl.ds(sid * zrows + k * CB, CB)])

            def fill(r, _):
                ones_v[r, pl.ds(0, L)] = jnp.full((L,), 1.0, jnp.float32)
                return _
            lax.fori_loop(0, CB, fill, 0)

        plsc.subcore_barrier()


        def convert(b):
            def conv(r, _):
                for u in range(2):
                    for k in range(d // 32):
                        v = rows_g[b, 2 * r + u, pl.ds(k * L, L)]
                        lo = plsc.bitcast(lax.shift_left(v, 16), jnp.float32)
                        hi = plsc.bitcast(
                            lax.bitwise_and(v, jnp.int32(-65536)), jnp.float32)
                        rows[2 * r + u, pl.ds(k * L, L)] = lo
                        rows[2 * r + u, pl.ds(d // 2 + k * L, L)] = hi
                return _
            lax.fori_loop(0, CB // 2, conv, 0)

        gat_sems = (sem_a, sem_b)

        def group(g, _):
            cb = base + g * grp
            pltpu.sync_copy(src_hbm.at[pl.ds(cb, grp)], srcv)
            pltpu.sync_copy(dst_hbm.at[pl.ds(cb, grp)], dstv)
            gat = {0: pltpu.async_copy(x_hbm.at[srcv.at[0]], rows_g.at[0], sem_a)}
            for j in range(grp):
                b = j & 1
                if j + 1 < grp:
                    gat[j + 1] = pltpu.async_copy(
                        x_hbm.at[srcv.at[j + 1]], rows_g.at[1 - b],
                        gat_sems[1 - b])
                gat[j].wait()
                convert(b)
                pltpu.sync_copy(rows, aggs.at[dstv.at[j]], add=True)
                if with_deg:
                    pltpu.sync_copy(ones_v, degs.at[dstv.at[j]], add=True)
            return _
        lax.fori_loop(0, ngroups, group, 0)

        plsc.subcore_barrier()

        for k in range(zrows // CB):
            off = sid * zrows + k * CB
            pltpu.sync_copy(aggs.at[pl.ds(off, CB)], rows)
            pltpu.sync_copy(rows, agg_out.at[cid, pl.ds(off, CB)])
        if with_deg:
            for k in range(zrows // CB):
                off = sid * zrows + k * CB
                pltpu.sync_copy(degs.at[pl.ds(off, CB)], ones_v)
                pltpu.sync_copy(ones_v, deg_out.at[cid, pl.ds(off, CB)])

    return pl.kernel(
        body, out_type=out_type, mesh=mesh, scratch_types=scratch,
        compiler_params=pltpu.CompilerParams(
            use_tc_tiling_on_sc=False, needs_layout_passes=False))


def _pack_bf16(x):
    d = x.shape[1]
    lo = lax.bitcast_convert_type(x[:, :d // 2].astype(jnp.bfloat16), jnp.uint16)
    hi = lax.bitcast_convert_type(x[:, d // 2:].astype(jnp.bfloat16), jnp.uint16)
    return (hi.astype(jnp.int32) << 16) | lo.astype(jnp.int32)


def _tc_layer(agg, deg, x, wt, wb, b):
    n, d = x.shape

    def body(agg_ref, deg_ref, x_ref, wt_ref, wb_ref, b_ref, o_ref, op_ref):
        s = agg_ref[0, :n] + agg_ref[1, :n]
        cnt = deg_ref[0, :n, 0:1] + deg_ref[1, :n, 0:1]
        s = s / jnp.maximum(cnt, 1.0)
        h = (jnp.dot(s, wt_ref[...], preferred_element_type=jnp.float32)
             + jnp.dot(x_ref[...], wb_ref[...], preferred_element_type=jnp.float32)
             + b_ref[...])
        h = jnp.maximum(h, 0.0)
        nrm = jnp.sqrt(jnp.sum(h * h, axis=1, keepdims=True))
        h = h / jnp.maximum(nrm, 1e-12)
        h = jnp.maximum(h, 0.0)
        o_ref[...] = h
        lo = lax.bitcast_convert_type(
            h[:, :d // 2].astype(jnp.bfloat16), jnp.uint16)
        hi = lax.bitcast_convert_type(
            h[:, d // 2:].astype(jnp.bfloat16), jnp.uint16)
        op_ref[...] = (hi.astype(jnp.int32) << 16) | lo.astype(jnp.int32)

    return pl.pallas_call(
        body,
        out_shape=[jax.ShapeDtypeStruct((n, d), jnp.float32),
                   jax.ShapeDtypeStruct((n, d // 2), jnp.int32)],
    )(agg, deg, x, wt, wb, b.reshape(1, -1))


def _tc_final(agg, deg, x, wt, wb, b, wp1, bp1, wp2, bp2):
    n, d = x.shape
    d_out = wp2.shape[1]

    def body(agg_ref, deg_ref, x_ref, wt_ref, wb_ref, b_ref,
             wp1_ref, bp1_ref, wp2_ref, bp2_ref, o_ref):
        s = agg_ref[0, :n] + agg_ref[1, :n]
        cnt = deg_ref[0, :n, 0:1] + deg_ref[1, :n, 0:1]
        s = s / jnp.maximum(cnt, 1.0)
        h = (jnp.dot(s, wt_ref[...], preferred_element_type=jnp.float32)
             + jnp.dot(x_ref[...], wb_ref[...], preferred_element_type=jnp.float32)
             + b_ref[...])
        h = jnp.maximum(h, 0.0)
        nrm = jnp.sqrt(jnp.sum(h * h, axis=1, keepdims=True))
        h = h / jnp.maximum(nrm, 1e-12)
        h = jnp.maximum(h, 0.0)
        p = jnp.dot(h, wp1_ref[...], preferred_element_type=jnp.float32) + bp1_ref[...]
        q = jnp.dot(p, wp2_ref[...], preferred_element_type=jnp.float32) + bp2_ref[...]
        m = jnp.max(q, axis=1, keepdims=True)
        e = jnp.exp(q - m)
        lse = jnp.log(jnp.sum(e, axis=1, keepdims=True))
        o_ref[...] = q - m - lse

    return pl.pallas_call(
        body,
        out_shape=jax.ShapeDtypeStruct((n, d_out), jnp.float32),
    )(agg, deg, x, wt, wb, b.reshape(1, -1),
      wp1, bp1.reshape(1, -1), wp2, bp2.reshape(1, -1))


def kernel(x, edge_index, batch, W0, b0, W1, b1, Wp1, bp1, Wp2, bp2):
    n, d = x.shape
    e = edge_index.shape[1]
    pair_chunks = -(-(-(-e // (NS * CB))) // 8) * 8
    epad = NS * pair_chunks * CB
    c0 = (pair_chunks * SHARE0_NUM // SHARE0_DEN) // 8 * 8

    src = edge_index[0]
    dst = edge_index[1]
    npad, _ = _acc_rows(n)
    pad_dst = n + jnp.arange(epad - e, dtype=jnp.int32) % (npad - n)
    srcp = jnp.concatenate(
        [src, jnp.zeros((epad - e,), jnp.int32)]).reshape(-1, CB)
    dstp = jnp.concatenate([dst, pad_dst]).reshape(-1, CB)

    agg_deg = _sc_aggregate(n, d, pair_chunks, c0, True)
    agg_only = _sc_aggregate(n, d, pair_chunks, c0, False)

    agg0, deg = agg_deg(_pack_bf16(x), srcp, dstp)
    h1, h1p = _tc_layer(agg0, deg, x, W0[:d], W0[d:], b0)
    (agg1,) = agg_only(h1p, srcp, dstp)
    return _tc_final(agg1, deg, h1, W1[:d], W1[d:], b1, Wp1, bp1, Wp2, bp2)
